# baseline (speedup 1.0000x reference)
import functools

import jax
import jax.numpy as jnp
from jax import lax
from jax.experimental import pallas as pl
from jax.experimental.pallas import tpu as pltpu
from jax.experimental.pallas import tpu_sc as plsc

NS = 4
NC = 1024
SD = 128
TOKENS = 9216
TBLK = 2304

_NUM_CORES = 2
_NUM_SUBCORES = 16
_NW = _NUM_CORES * _NUM_SUBCORES
_B = TOKENS * NS
_B_PER_W = _B // _NW
_T_PER_W = _B_PER_W // NS
_CHUNK = 288
_NCHUNK = _B_PER_W // _CHUNK


_HALF = TOKENS // 2


def _dist_body(x_ref, cb_ref, nn_ref, nng_ref, loss_ref):
    x = x_ref[...]
    acc = jnp.zeros((TBLK,), jnp.float32)
    for s in range(NS):
        xs = x[:, s * SD:(s + 1) * SD]
        cbs = cb_ref[s]
        cb2 = cbs + cbs
        mm2 = lax.dot_general(xs, cb2, (((1,), (1,)), ((), ())),
                              preferred_element_type=jnp.float32)
        xn = jnp.sum(xs * xs, axis=1)
        cn = jnp.sum(cbs * cbs, axis=1)
        d = (xn[:, None] - mm2) + cn[None, :]
        m = jnp.min(d, axis=1)
        iota = lax.broadcasted_iota(jnp.int32, d.shape, 1)
        idx = jnp.min(jnp.where(d == m[:, None], iota, jnp.int32(NC)), axis=1)
        nn_ref[s, :] = idx
        nng_ref[s, :] = idx + jnp.int32(s * NC)
        acc = acc + m
    loss_ref[0, :] = acc * (1.0 / (NS * SD))


def _distances(flat, codebooks, ntok):
    grid = (ntok // TBLK,)
    return pl.pallas_call(
        _dist_body,
        grid=grid,
        in_specs=[
            pl.BlockSpec((TBLK, NS * SD), lambda i: (i, 0)),
            pl.BlockSpec((NS, NC, SD), lambda i: (0, 0, 0)),
        ],
        out_specs=[
            pl.BlockSpec((NS, TBLK), lambda i: (0, i)),
            pl.BlockSpec((NS, TBLK), lambda i: (0, i)),
            pl.BlockSpec((1, TBLK), lambda i: (0, i)),
        ],
        out_shape=[
            jax.ShapeDtypeStruct((NS, ntok), jnp.int32),
            jax.ShapeDtypeStruct((NS, ntok), jnp.int32),
            jax.ShapeDtypeStruct((1, ntok), jnp.float32),
        ],
    )(flat, codebooks)


@functools.cache
def _make_sc_gather(ntok):
    t_per_w = ntok // _NW
    mesh = plsc.VectorSubcoreMesh(core_axis_name="c", subcore_axis_name="s")

    @functools.partial(
        pl.kernel,
        mesh=mesh,
        out_type=jax.ShapeDtypeStruct((ntok, NS, SD), jnp.float32),
        scratch_types=[
            pltpu.VMEM((t_per_w,), jnp.int32),
            pltpu.VMEM((t_per_w,), jnp.int32),
            pltpu.VMEM((t_per_w,), jnp.int32),
            pltpu.VMEM((t_per_w,), jnp.int32),
            pltpu.VMEM((t_per_w, 1, SD), jnp.float32),
            pltpu.VMEM((t_per_w, 1, SD), jnp.float32),
            pltpu.SemaphoreType.DMA,
            pltpu.SemaphoreType.DMA,
        ],
    )
    def _sc_gather(table_hbm, nng_hbm, out_hbm, gidx0, gidx1, gidx2, gidx3,
                   buf0, buf1, sem0, sem1):
        gidx = (gidx0, gidx1, gidx2, gidx3)
        wid = lax.axis_index("s") * _NUM_CORES + lax.axis_index("c")
        t0 = wid * t_per_w
        for s in range(NS):
            pltpu.sync_copy(nng_hbm.at[pl.ds(s * ntok + t0, t_per_w)],
                            gidx[s])
        bufs = (buf0, buf1)
        sems = (sem0, sem1)
        handles = [None, None]
        for s in range(NS):
            handles[s % 2] = pltpu.async_copy(
                table_hbm.at[gidx[s]], bufs[s % 2], sems[s % 2])
            if s >= 1:
                handles[(s - 1) % 2].wait()
                pltpu.sync_copy(
                    bufs[(s - 1) % 2],
                    out_hbm.at[pl.ds(t0, t_per_w), pl.ds(s - 1, 1)])
        handles[(NS - 1) % 2].wait()
        pltpu.sync_copy(bufs[(NS - 1) % 2],
                        out_hbm.at[pl.ds(t0, t_per_w), pl.ds(NS - 1, 1)])

    return _sc_gather


def kernel(inputs, codebooks, train):
    flat = jnp.reshape(inputs, (-1, NS * SD))
    table = jnp.reshape(codebooks, (NS * NC, 1, SD))
    sc = _make_sc_gather(_HALF)
    nn_h, loss_h, q_h = [], [], []
    for h in range(2):
        fh = lax.slice_in_dim(flat, h * _HALF, (h + 1) * _HALF, axis=0)
        nn, nng, loss = _distances(fh, codebooks, _HALF)
        nng_flat = jnp.reshape(nng, (_HALF * NS,))
        q_h.append(sc(table, nng_flat))
        nn_h.append(nn)
        loss_h.append(loss)
    nn = jnp.concatenate(nn_h, axis=1)
    loss = jnp.concatenate(loss_h, axis=1)
    gathered = jnp.concatenate(q_h, axis=0)
    quantized = jnp.reshape(gathered, inputs.shape)
    qloss = jnp.reshape(loss, inputs.shape[:-1] + (1,))
    nn_out = jnp.reshape(nn, (NS,) + inputs.shape[:-1])
    codebook = jnp.reshape(codebooks, (NS * NC, SD))
    return quantized, qloss, nn_out, codebook


# speedup vs baseline: 1.3488x; 1.3488x over previous
import functools

import jax
import jax.numpy as jnp
from jax import lax
from jax.experimental import pallas as pl
from jax.experimental.pallas import tpu as pltpu
from jax.experimental.pallas import tpu_sc as plsc

NS = 4
NC = 1024
SD = 128
TOKENS = 9216
TBLK = 3072

_NUM_CORES = 2
_NUM_SUBCORES = 16
_NW = _NUM_CORES * _NUM_SUBCORES
_B = TOKENS * NS
_B_PER_W = _B // _NW
_T_PER_W = _B_PER_W // NS
_CHUNK = 288
_NCHUNK = _B_PER_W // _CHUNK


_HALF = TOKENS // 2


def _dist_body(x_ref, cb_ref, nn_ref, nng_ref, loss_ref):
    x = x_ref[...]
    acc = jnp.zeros((TBLK,), jnp.float32)
    for s in range(NS):
        xs = x[:, s * SD:(s + 1) * SD]
        cbs = cb_ref[s]
        cb2 = cbs + cbs
        mm2 = lax.dot_general(xs, cb2, (((1,), (1,)), ((), ())),
                              preferred_element_type=jnp.float32)
        xn = jnp.sum(xs * xs, axis=1)
        cn = jnp.sum(cbs * cbs, axis=1)
        d = (xn[:, None] - mm2) + cn[None, :]
        m = jnp.min(d, axis=1)
        iota = lax.broadcasted_iota(jnp.int32, d.shape, 1)
        idx = jnp.min(jnp.where(d == m[:, None], iota, jnp.int32(NC)), axis=1)
        nn_ref[s, :] = idx
        nng_ref[s, :] = idx + jnp.int32(s * NC)
        acc = acc + m
    loss_ref[0, :] = acc * (1.0 / (NS * SD))


def _distances(flat, codebooks, ntok):
    grid = (ntok // TBLK,)
    return pl.pallas_call(
        _dist_body,
        grid=grid,
        in_specs=[
            pl.BlockSpec((TBLK, NS * SD), lambda i: (i, 0)),
            pl.BlockSpec((NS, NC, SD), lambda i: (0, 0, 0)),
        ],
        out_specs=[
            pl.BlockSpec((NS, TBLK), lambda i: (0, i)),
            pl.BlockSpec((NS, TBLK), lambda i: (0, i)),
            pl.BlockSpec((1, TBLK), lambda i: (0, i)),
        ],
        out_shape=[
            jax.ShapeDtypeStruct((NS, ntok), jnp.int32),
            jax.ShapeDtypeStruct((NS, ntok), jnp.int32),
            jax.ShapeDtypeStruct((1, ntok), jnp.float32),
        ],
    )(flat, codebooks)


@functools.cache
def _make_sc_gather(ntok):
    t_per_w = ntok // _NW
    mesh = plsc.VectorSubcoreMesh(core_axis_name="c", subcore_axis_name="s")

    @functools.partial(
        pl.kernel,
        mesh=mesh,
        out_type=jax.ShapeDtypeStruct((ntok, NS, SD), jnp.float32),
        scratch_types=[
            pltpu.VMEM((t_per_w,), jnp.int32),
            pltpu.VMEM((t_per_w,), jnp.int32),
            pltpu.VMEM((t_per_w,), jnp.int32),
            pltpu.VMEM((t_per_w,), jnp.int32),
            pltpu.VMEM((t_per_w, 1, SD), jnp.float32),
            pltpu.VMEM((t_per_w, 1, SD), jnp.float32),
            pltpu.SemaphoreType.DMA,
            pltpu.SemaphoreType.DMA,
        ],
    )
    def _sc_gather(table_hbm, nng_hbm, out_hbm, gidx0, gidx1, gidx2, gidx3,
                   buf0, buf1, sem0, sem1):
        gidx = (gidx0, gidx1, gidx2, gidx3)
        wid = lax.axis_index("s") * _NUM_CORES + lax.axis_index("c")
        t0 = wid * t_per_w
        for s in range(NS):
            pltpu.sync_copy(nng_hbm.at[pl.ds(s * ntok + t0, t_per_w)],
                            gidx[s])
        bufs = (buf0, buf1)
        sems = (sem0, sem1)
        handles = [None, None]
        for s in range(NS):
            handles[s % 2] = pltpu.async_copy(
                table_hbm.at[gidx[s]], bufs[s % 2], sems[s % 2])
            if s >= 1:
                handles[(s - 1) % 2].wait()
                pltpu.sync_copy(
                    bufs[(s - 1) % 2],
                    out_hbm.at[pl.ds(t0, t_per_w), pl.ds(s - 1, 1)])
        handles[(NS - 1) % 2].wait()
        pltpu.sync_copy(bufs[(NS - 1) % 2],
                        out_hbm.at[pl.ds(t0, t_per_w), pl.ds(NS - 1, 1)])

    return _sc_gather


def kernel(inputs, codebooks, train):
    flat = jnp.reshape(inputs, (-1, NS * SD))
    table = jnp.reshape(codebooks, (NS * NC, 1, SD))
    nn, nng, loss = _distances(flat, codebooks, TOKENS)
    nng_flat = jnp.reshape(nng, (_B,))
    gathered = _make_sc_gather(TOKENS)(table, nng_flat)
    quantized = jnp.reshape(gathered, inputs.shape)
    qloss = jnp.reshape(loss, inputs.shape[:-1] + (1,))
    nn_out = jnp.reshape(nn, (NS,) + inputs.shape[:-1])
    codebook = jnp.reshape(codebooks, (NS * NC, SD))
    return quantized, qloss, nn_out, codebook


# speedup vs baseline: 1.3571x; 1.0062x over previous
import functools

import jax
import jax.numpy as jnp
from jax import lax
from jax.experimental import pallas as pl
from jax.experimental.pallas import tpu as pltpu
from jax.experimental.pallas import tpu_sc as plsc

NS = 4
NC = 1024
SD = 128
TOKENS = 9216
TBLK = 3072

_NUM_CORES = 2
_NUM_SUBCORES = 16
_NW = _NUM_CORES * _NUM_SUBCORES
_B = TOKENS * NS
_B_PER_W = _B // _NW
_T_PER_W = _B_PER_W // NS
_CHUNK = 288
_NCHUNK = _B_PER_W // _CHUNK


_HALF = TOKENS // 2


def _dist_body(x_ref, cb_ref, nn_ref, nng_ref, loss_ref):
    x = x_ref[...]
    acc = jnp.zeros((TBLK,), jnp.float32)
    for s in range(NS):
        xs = x[:, s * SD:(s + 1) * SD]
        cbs = cb_ref[s]
        cb2 = cbs + cbs
        mm2 = lax.dot_general(xs, cb2, (((1,), (1,)), ((), ())),
                              preferred_element_type=jnp.float32)
        xn = jnp.sum(xs * xs, axis=1)
        cn = jnp.sum(cbs * cbs, axis=1)
        d = (xn[:, None] - mm2) + cn[None, :]
        m = jnp.min(d, axis=1)
        fiota = lax.broadcasted_iota(jnp.int32, d.shape, 1).astype(jnp.float32)
        key = (d - m[:, None]) * jnp.float32(3e38) + fiota
        idx = jnp.min(key, axis=1).astype(jnp.int32)
        nn_ref[s, :] = idx
        nng_ref[s, :] = idx + jnp.int32(s * NC)
        acc = acc + m
    loss_ref[0, :] = acc * (1.0 / (NS * SD))


def _distances(flat, codebooks, ntok):
    grid = (ntok // TBLK,)
    return pl.pallas_call(
        _dist_body,
        grid=grid,
        in_specs=[
            pl.BlockSpec((TBLK, NS * SD), lambda i: (i, 0)),
            pl.BlockSpec((NS, NC, SD), lambda i: (0, 0, 0)),
        ],
        out_specs=[
            pl.BlockSpec((NS, TBLK), lambda i: (0, i)),
            pl.BlockSpec((NS, TBLK), lambda i: (0, i)),
            pl.BlockSpec((1, TBLK), lambda i: (0, i)),
        ],
        out_shape=[
            jax.ShapeDtypeStruct((NS, ntok), jnp.int32),
            jax.ShapeDtypeStruct((NS, ntok), jnp.int32),
            jax.ShapeDtypeStruct((1, ntok), jnp.float32),
        ],
    )(flat, codebooks)


@functools.cache
def _make_sc_gather(ntok):
    t_per_w = ntok // _NW
    mesh = plsc.VectorSubcoreMesh(core_axis_name="c", subcore_axis_name="s")

    @functools.partial(
        pl.kernel,
        mesh=mesh,
        out_type=jax.ShapeDtypeStruct((ntok, NS, SD), jnp.float32),
        scratch_types=[
            pltpu.VMEM((t_per_w,), jnp.int32),
            pltpu.VMEM((t_per_w,), jnp.int32),
            pltpu.VMEM((t_per_w,), jnp.int32),
            pltpu.VMEM((t_per_w,), jnp.int32),
            pltpu.VMEM((t_per_w, 1, SD), jnp.float32),
            pltpu.VMEM((t_per_w, 1, SD), jnp.float32),
            pltpu.SemaphoreType.DMA,
            pltpu.SemaphoreType.DMA,
        ],
    )
    def _sc_gather(table_hbm, nng_hbm, out_hbm, gidx0, gidx1, gidx2, gidx3,
                   buf0, buf1, sem0, sem1):
        gidx = (gidx0, gidx1, gidx2, gidx3)
        wid = lax.axis_index("s") * _NUM_CORES + lax.axis_index("c")
        t0 = wid * t_per_w
        for s in range(NS):
            pltpu.sync_copy(nng_hbm.at[pl.ds(s * ntok + t0, t_per_w)],
                            gidx[s])
        bufs = (buf0, buf1)
        sems = (sem0, sem1)
        handles = [None, None]
        for s in range(NS):
            handles[s % 2] = pltpu.async_copy(
                table_hbm.at[gidx[s]], bufs[s % 2], sems[s % 2])
            if s >= 1:
                handles[(s - 1) % 2].wait()
                pltpu.sync_copy(
                    bufs[(s - 1) % 2],
                    out_hbm.at[pl.ds(t0, t_per_w), pl.ds(s - 1, 1)])
        handles[(NS - 1) % 2].wait()
        pltpu.sync_copy(bufs[(NS - 1) % 2],
                        out_hbm.at[pl.ds(t0, t_per_w), pl.ds(NS - 1, 1)])

    return _sc_gather


def kernel(inputs, codebooks, train):
    flat = jnp.reshape(inputs, (-1, NS * SD))
    table = jnp.reshape(codebooks, (NS * NC, 1, SD))
    nn, nng, loss = _distances(flat, codebooks, TOKENS)
    nng_flat = jnp.reshape(nng, (_B,))
    gathered = _make_sc_gather(TOKENS)(table, nng_flat)
    quantized = jnp.reshape(gathered, inputs.shape)
    qloss = jnp.reshape(loss, inputs.shape[:-1] + (1,))
    nn_out = jnp.reshape(nn, (NS,) + inputs.shape[:-1])
    codebook = jnp.reshape(codebooks, (NS * NC, SD))
    return quantized, qloss, nn_out, codebook
